# Initial kernel scaffold; baseline (speedup 1.0000x reference)
import jax
import jax.numpy as jnp
from jax.experimental import pallas as pl


def kernel(x_nchw, conv_w_0, conv_b_0, conv_w_1, conv_b_1, conv_w_2, conv_b_2, conv_w_3, conv_b_3, conv_w_4, conv_b_4, conv_w_5, conv_b_5, conv_w_6, conv_b_6, conv_w_7, conv_b_7, conv_w_8, conv_b_8, conv_w_9, conv_b_9, conv_w_10, conv_b_10, conv_w_11, conv_b_11, conv_w_12, conv_b_12, fc_w_0, fc_b_0, fc_w_1, fc_b_1):
    raise NotImplementedError("write your pallas kernel here")


# speedup vs baseline: 15.1931x; 15.1931x over previous
import functools

import jax
import jax.numpy as jnp
import numpy as np
from jax.experimental import pallas as pl
from jax.experimental.pallas import tpu as pltpu

LANE = 128

_LAYERS = [
    (32, 32, 4, 4, 4, False),
    (32, 32, 4, 4, 4, True),
    (16, 16, 4, 8, 8, False),
    (16, 16, 8, 8, 8, True),
    (8, 8, 8, 16, 16, False),
    (8, 8, 16, 16, 16, False),
    (8, 8, 16, 16, 16, True),
    (4, 4, 16, 32, 32, False),
    (4, 4, 32, 32, 32, False),
    (4, 4, 32, 32, 32, True),
    (2, 2, 32, 64, 32, False),
    (2, 2, 32, 32, 32, False),
    (2, 2, 32, 32, 32, True),
]


def _banded_indices(Wi, Ci, s_in, Co):
    w, k, i, o = np.meshgrid(np.arange(Wi), np.arange(3), np.arange(Ci),
                             np.arange(Co), indexing="ij")
    valid = ((w + k - 1) >= 0) & ((w + k - 1) < Wi)
    w, k, i, o = w[valid], k[valid], i[valid], o[valid]
    pin = (w + k - 1) * s_in + i
    qout = w * Co + o
    return k, i, o, pin, qout


def _fused_kernel(x_ref, w_ref, b_ref, o_ref, *, nb):
    x = x_ref[...]
    H = 32
    for l, (_, Wi, Ci, s_in, Co, pool) in enumerate(_LAYERS):
        xp = jnp.pad(x, ((1, 1), (0, 0), (0, 0)))
        acc = None
        for kh in range(3):
            xs = xp[kh:kh + H].reshape(H * nb, LANE)
            part = jnp.dot(xs, w_ref[3 * l + kh],
                           preferred_element_type=jnp.float32)
            acc = part if acc is None else acc + part
        y = jnp.maximum(acc + b_ref[l:l + 1, :], 0.0)
        y = y.reshape(H, nb, LANE)
        if pool:
            H //= 2
            y = y.reshape(H, 2, nb, LANE).max(axis=1)
            ysh = jnp.pad(y[:, :, Co:], ((0, 0), (0, 0), (0, Co)))
            y = jnp.maximum(y, ysh)
        x = y.astype(jnp.bfloat16)

    a = x.reshape(nb, LANE)
    lane = jax.lax.broadcasted_iota(jnp.int32, (nb, LANE), 1)
    a = jnp.where(lane < 32, a, jnp.zeros_like(a))
    h = jnp.dot(a, w_ref[39], preferred_element_type=jnp.float32)
    h = jnp.maximum(h + b_ref[13:14, :], 0.0).astype(jnp.bfloat16)
    h = jnp.dot(h, w_ref[40], preferred_element_type=jnp.float32)
    o_ref[...] = jnp.maximum(h + b_ref[14:15, :], 0.0)


def kernel(x_nchw, conv_w_0, conv_b_0, conv_w_1, conv_b_1, conv_w_2, conv_b_2,
           conv_w_3, conv_b_3, conv_w_4, conv_b_4, conv_w_5, conv_b_5,
           conv_w_6, conv_b_6, conv_w_7, conv_b_7, conv_w_8, conv_b_8,
           conv_w_9, conv_b_9, conv_w_10, conv_b_10, conv_w_11, conv_b_11,
           conv_w_12, conv_b_12, fc_w_0, fc_b_0, fc_w_1, fc_b_1):
    conv_w = [conv_w_0, conv_w_1, conv_w_2, conv_w_3, conv_w_4, conv_w_5,
              conv_w_6, conv_w_7, conv_w_8, conv_w_9, conv_w_10, conv_w_11,
              conv_w_12]
    conv_b = [conv_b_0, conv_b_1, conv_b_2, conv_b_3, conv_b_4, conv_b_5,
              conv_b_6, conv_b_7, conv_b_8, conv_b_9, conv_b_10, conv_b_11,
              conv_b_12]

    N = x_nchw.shape[0]
    nb = min(128, N)
    assert N % nb == 0

    x = jnp.transpose(x_nchw, (2, 0, 3, 1))
    x = jnp.pad(x, ((0, 0), (0, 0), (0, 0), (0, 1)))
    x = x.reshape(32, N, LANE).astype(jnp.bfloat16)

    Bs, bs = [], []
    for l, (_, Wi, Ci, s_in, Co, _pool) in enumerate(_LAYERS):
        k, i, o, pin, qout = _banded_indices(Wi, Ci, s_in, Co)
        B = jnp.zeros((3, LANE, LANE), conv_w[l].dtype)
        B = B.at[:, pin, qout].set(conv_w[l][:, k, i, o])
        Bs.append(B)
        bs.append(jnp.pad(jnp.tile(conv_b[l][:Co], Wi), (0, LANE - Wi * Co)))
    w_all = jnp.concatenate(
        Bs + [fc_w_0[None].astype(jnp.bfloat16),
              fc_w_1[None].astype(jnp.bfloat16)], axis=0)
    b_all = jnp.stack(bs + [fc_b_0.astype(jnp.float32),
                            fc_b_1.astype(jnp.float32),
                            jnp.zeros((LANE,), jnp.float32)])

    flops = 2 * N * (sum(3 * h * LANE * LANE for (h, *_r) in _LAYERS)
                     + 2 * LANE * LANE)
    bytes_accessed = x.size * 2 + w_all.size * 2 + b_all.size * 4 + N * LANE * 4

    return pl.pallas_call(
        functools.partial(_fused_kernel, nb=nb),
        out_shape=jax.ShapeDtypeStruct((N, LANE), jnp.float32),
        grid=(N // nb,),
        in_specs=[
            pl.BlockSpec((32, nb, LANE), lambda n: (0, n, 0)),
            pl.BlockSpec((41, LANE, LANE), lambda n: (0, 0, 0)),
            pl.BlockSpec((16, LANE), lambda n: (0, 0)),
        ],
        out_specs=pl.BlockSpec((nb, LANE), lambda n: (n, 0)),
        compiler_params=pltpu.CompilerParams(
            dimension_semantics=("parallel",),
            vmem_limit_bytes=48 * 1024 * 1024),
        cost_estimate=pl.CostEstimate(flops=int(flops), transcendentals=0,
                                      bytes_accessed=int(bytes_accessed)),
    )(x, w_all, b_all)
